# baseline (speedup 1.0000x reference)
import functools

import jax
import jax.numpy as jnp
from jax import lax
from jax.experimental import pallas as pl
from jax.experimental.pallas import tpu as pltpu
from jax.experimental.pallas import tpu_sc as plsc

_NUM_CLASSES = 19
_BH = 128
_NC, _NS, _L = 2, 16, 16
_NW = _NC * _NS
_CROWS = 32


def _argmax_body(x_ref, o_ref):
    x = x_ref[0]
    m = x[0]
    idx = jnp.zeros(m.shape, dtype=jnp.int32)
    for c in range(1, _NUM_CLASSES):
        v = x[c]
        gt = v > m
        m = jnp.where(gt, v, m)
        idx = jnp.where(gt, c, idx)
    o_ref[0] = idx


def _tc_argmax(soft_label, b0, nb):
    b, nc, h, wdim = soft_label.shape
    return pl.pallas_call(
        _argmax_body,
        grid=(nb, h // _BH),
        in_specs=[pl.BlockSpec((1, nc, _BH, wdim),
                               lambda i, j: (b0 + i, 0, j, 0))],
        out_specs=pl.BlockSpec((1, _BH, wdim), lambda i, j: (i, j, 0)),
        out_shape=jax.ShapeDtypeStruct((nb, h, wdim), jnp.int32),
    )(soft_label)


def _make_sc_gather(b, h, wdim):
    rows_per_w = (b * h) // _NW
    n_chunks = rows_per_w // _CROWS
    vecs = (_CROWS * wdim) // _L
    cols = wdim // _L
    mesh = plsc.VectorSubcoreMesh(
        core_axis_name="c", subcore_axis_name="s",
        num_cores=_NC, num_subcores=_NS)

    @functools.partial(
        pl.kernel,
        mesh=mesh,
        compiler_params=pltpu.CompilerParams(
            needs_layout_passes=False, use_tc_tiling_on_sc=True),
        out_type=jax.ShapeDtypeStruct((b, h, wdim), jnp.float32),
        scratch_types=[
            pltpu.VMEM((_NUM_CLASSES,), jnp.float32),
            pltpu.VMEM((_CROWS, wdim), jnp.int32),
            pltpu.VMEM((_CROWS, wdim), jnp.float32),
        ],
    )
    def _sc_gather(tbl_hbm, idx_hbm, out_hbm, tbl_v, idx_v, out_v):
        wid = lax.axis_index("s") * _NC + lax.axis_index("c")
        row_w = wid * rows_per_w
        rows_per_b = h

        def chunk_body(ci, _):
            row = row_w + ci * _CROWS
            bb = row // rows_per_b
            rr = row % rows_per_b
            pltpu.sync_copy(idx_hbm.at[bb, pl.ds(rr, _CROWS)], idx_v)

            @plsc.parallel_loop(0, vecs, unroll=8)
            def vec_body(i):
                r = i // cols
                c = (i % cols) * _L
                iv = idx_v[r, pl.ds(c, _L)]
                out_v[r, pl.ds(c, _L)] = plsc.load_gather(tbl_v, [iv])

            pltpu.sync_copy(out_v, out_hbm.at[bb, pl.ds(rr, _CROWS)])
            return 0

        pltpu.sync_copy(tbl_hbm, tbl_v)
        lax.fori_loop(0, n_chunks, chunk_body, 0)

    return _sc_gather


_K = 4


@jax.jit
def kernel(soft_label, pixel_weights):
    b, nc, h, wdim = soft_label.shape
    nb = b // _K
    gather = _make_sc_gather(nb, h, wdim)
    outs = []
    for k in range(_K):
        idx_k = _tc_argmax(soft_label, k * nb, nb)
        outs.append(gather(pixel_weights, idx_k))
    return jnp.concatenate(outs, axis=0)


# speedup vs baseline: 1.0793x; 1.0793x over previous
import functools

import jax
import jax.numpy as jnp
from jax import lax
from jax.experimental import pallas as pl
from jax.experimental.pallas import tpu as pltpu
from jax.experimental.pallas import tpu_sc as plsc

_NUM_CLASSES = 19
_BH = 128
_NC, _NS, _L = 2, 16, 16
_NW = _NC * _NS
_CROWS = 32


def _argmax_body(x_ref, o_ref):
    x = x_ref[0]
    m = x[0]
    idx = jnp.zeros(m.shape, dtype=jnp.int32)
    for c in range(1, _NUM_CLASSES):
        v = x[c]
        gt = v > m
        m = jnp.where(gt, v, m)
        idx = jnp.where(gt, c, idx)
    o_ref[0] = idx


def _tc_argmax(soft_label, b0, nb):
    b, nc, h, wdim = soft_label.shape
    return pl.pallas_call(
        _argmax_body,
        grid=(nb, h // _BH),
        in_specs=[pl.BlockSpec((1, nc, _BH, wdim),
                               lambda i, j: (b0 + i, 0, j, 0))],
        out_specs=pl.BlockSpec((1, _BH, wdim), lambda i, j: (i, j, 0)),
        out_shape=jax.ShapeDtypeStruct((nb, h, wdim), jnp.int32),
    )(soft_label)


def _make_sc_gather(nb, h, wdim, b0):
    rows_per_w = (nb * h) // _NW
    n_chunks = rows_per_w // _CROWS
    vecs = (_CROWS * wdim) // _L
    cols = wdim // _L
    mesh = plsc.VectorSubcoreMesh(
        core_axis_name="c", subcore_axis_name="s",
        num_cores=_NC, num_subcores=_NS)

    @functools.partial(
        pl.kernel,
        mesh=mesh,
        compiler_params=pltpu.CompilerParams(
            needs_layout_passes=False, use_tc_tiling_on_sc=True),
        out_type=(),
        scratch_types=[
            pltpu.VMEM((_NUM_CLASSES,), jnp.float32),
            pltpu.VMEM((_CROWS, wdim), jnp.int32),
            pltpu.VMEM((_CROWS, wdim), jnp.float32),
        ],
    )
    def _sc_gather(tbl_hbm, idx_hbm, out_hbm, tbl_v, idx_v, out_v):
        wid = lax.axis_index("s") * _NC + lax.axis_index("c")
        row_w = wid * rows_per_w

        def chunk_body(ci, _):
            row = row_w + ci * _CROWS
            bb = row // h
            rr = row % h
            pltpu.sync_copy(idx_hbm.at[bb, pl.ds(rr, _CROWS)], idx_v)

            @plsc.parallel_loop(0, vecs, unroll=8)
            def vec_body(i):
                r = i // cols
                c = (i % cols) * _L
                iv = idx_v[r, pl.ds(c, _L)]
                out_v[r, pl.ds(c, _L)] = plsc.load_gather(tbl_v, [iv])

            pltpu.sync_copy(out_v, out_hbm.at[b0 + bb, pl.ds(rr, _CROWS)])
            return 0

        pltpu.sync_copy(tbl_hbm, tbl_v)
        lax.fori_loop(0, n_chunks, chunk_body, 0)

    return _sc_gather


_K = 2


@jax.jit
def kernel(soft_label, pixel_weights):
    b, nc, h, wdim = soft_label.shape
    nb = b // _K
    out_ref = jax.new_ref(jnp.zeros((b, h, wdim), jnp.float32))
    for k in range(_K):
        idx_k = _tc_argmax(soft_label, k * nb, nb)
        _make_sc_gather(nb, h, wdim, k * nb)(pixel_weights, idx_k, out_ref)
    return jax.freeze(out_ref)


# speedup vs baseline: 1.0868x; 1.0070x over previous
import functools

import jax
import jax.numpy as jnp
from jax import lax
from jax.experimental import pallas as pl
from jax.experimental.pallas import tpu as pltpu
from jax.experimental.pallas import tpu_sc as plsc

_NUM_CLASSES = 19
_BH = 128
_NC, _NS, _L = 2, 16, 16
_NW = _NC * _NS
_CROWS = 64


def _argmax_body(x_ref, o_ref):
    x = x_ref[0]
    m = x[0]
    idx = jnp.zeros(m.shape, dtype=jnp.int32)
    for c in range(1, _NUM_CLASSES):
        v = x[c]
        gt = v > m
        m = jnp.where(gt, v, m)
        idx = jnp.where(gt, c, idx)
    o_ref[0] = idx


def _tc_argmax(soft_label, b0, nb):
    b, nc, h, wdim = soft_label.shape
    return pl.pallas_call(
        _argmax_body,
        grid=(nb, h // _BH),
        in_specs=[pl.BlockSpec((1, nc, _BH, wdim),
                               lambda i, j: (b0 + i, 0, j, 0))],
        out_specs=pl.BlockSpec((1, _BH, wdim), lambda i, j: (i, j, 0)),
        out_shape=jax.ShapeDtypeStruct((nb, h, wdim), jnp.int32),
    )(soft_label)


def _make_sc_gather(nb, h, wdim, b0):
    rows_per_w = (nb * h) // _NW
    n_chunks = rows_per_w // _CROWS
    vecs = (_CROWS * wdim) // _L
    cols = wdim // _L
    mesh = plsc.VectorSubcoreMesh(
        core_axis_name="c", subcore_axis_name="s",
        num_cores=_NC, num_subcores=_NS)

    @functools.partial(
        pl.kernel,
        mesh=mesh,
        compiler_params=pltpu.CompilerParams(
            needs_layout_passes=False, use_tc_tiling_on_sc=True),
        out_type=(),
        scratch_types=[
            pltpu.VMEM((_NUM_CLASSES,), jnp.float32),
            pltpu.VMEM((_CROWS, wdim), jnp.int32),
            pltpu.VMEM((_CROWS, wdim), jnp.float32),
        ],
    )
    def _sc_gather(tbl_hbm, idx_hbm, out_hbm, tbl_v, idx_v, out_v):
        wid = lax.axis_index("s") * _NC + lax.axis_index("c")
        row_w = wid * rows_per_w

        def chunk_body(ci, _):
            row = row_w + ci * _CROWS
            bb = row // h
            rr = row % h
            pltpu.sync_copy(idx_hbm.at[bb, pl.ds(rr, _CROWS)], idx_v)

            @plsc.parallel_loop(0, vecs, unroll=8)
            def vec_body(i):
                r = i // cols
                c = (i % cols) * _L
                iv = idx_v[r, pl.ds(c, _L)]
                out_v[r, pl.ds(c, _L)] = plsc.load_gather(tbl_v, [iv])

            pltpu.sync_copy(out_v, out_hbm.at[b0 + bb, pl.ds(rr, _CROWS)])
            return 0

        pltpu.sync_copy(tbl_hbm, tbl_v)
        lax.fori_loop(0, n_chunks, chunk_body, 0)

    return _sc_gather


_K = 2


@jax.jit
def kernel(soft_label, pixel_weights):
    b, nc, h, wdim = soft_label.shape
    nb = b // _K
    idxs = [_tc_argmax(soft_label, k * nb, nb) for k in range(_K)]
    out_ref = jax.new_ref(jnp.zeros((b, h, wdim), jnp.float32))
    for k in range(_K):
        _make_sc_gather(nb, h, wdim, k * nb)(pixel_weights, idxs[k], out_ref)
    return jax.freeze(out_ref)


# speedup vs baseline: 1.1792x; 1.0850x over previous
import functools

import jax
import jax.numpy as jnp
from jax import lax
from jax.experimental import pallas as pl
from jax.experimental.pallas import tpu as pltpu
from jax.experimental.pallas import tpu_sc as plsc

_NUM_CLASSES = 19
_BH = 128
_NC, _NS, _L = 2, 16, 16
_NW = _NC * _NS
_CROWS = 32


def _argmax_body(x_ref, o_ref):
    x = x_ref[0]
    m = x[0]
    idx = jnp.zeros(m.shape, dtype=jnp.int32)
    for c in range(1, _NUM_CLASSES):
        v = x[c]
        gt = v > m
        m = jnp.where(gt, v, m)
        idx = jnp.where(gt, c, idx)
    o_ref[0] = idx


def _tc_argmax(soft_label):
    b, nc, h, wdim = soft_label.shape
    return pl.pallas_call(
        _argmax_body,
        grid=(b, h // _BH),
        in_specs=[pl.BlockSpec((1, nc, _BH, wdim), lambda i, j: (i, 0, j, 0))],
        out_specs=pl.BlockSpec((1, _BH, wdim), lambda i, j: (i, j, 0)),
        out_shape=jax.ShapeDtypeStruct((b, h, wdim), jnp.int32),
    )(soft_label)


def _make_sc_gather(b, h, wdim):
    rows_per_w = (b * h) // _NW
    n_chunks = rows_per_w // _CROWS
    vecs = (_CROWS * wdim) // _L
    cols = wdim // _L
    mesh = plsc.VectorSubcoreMesh(
        core_axis_name="c", subcore_axis_name="s",
        num_cores=_NC, num_subcores=_NS)

    @functools.partial(
        pl.kernel,
        mesh=mesh,
        compiler_params=pltpu.CompilerParams(
            needs_layout_passes=False, use_tc_tiling_on_sc=True),
        out_type=jax.ShapeDtypeStruct((b, h, wdim), jnp.float32),
        scratch_types=[
            pltpu.VMEM((_NUM_CLASSES,), jnp.float32),
            pltpu.VMEM((2, _CROWS, wdim), jnp.int32),
            pltpu.VMEM((2, _CROWS, wdim), jnp.float32),
            pltpu.SemaphoreType.DMA((2,)),
            pltpu.SemaphoreType.DMA((2,)),
        ],
    )
    def _sc_gather(tbl_hbm, idx_hbm, out_hbm, tbl_v, idx_v, out_v,
                   in_sems, out_sems):
        wid = lax.axis_index("s") * _NC + lax.axis_index("c")
        row_w = wid * rows_per_w

        def slices(ci):
            row = row_w + ci * _CROWS
            return row // h, row % h

        pltpu.sync_copy(tbl_hbm, tbl_v)

        in_flight = []
        bb0, rr0 = slices(0)
        in_flight.append(pltpu.async_copy(
            idx_hbm.at[bb0, pl.ds(rr0, _CROWS)], idx_v.at[0], in_sems.at[0]))

        out_flight = [None, None]
        for ci in range(n_chunks):
            buf = ci % 2
            if ci + 1 < n_chunks:
                bb1, rr1 = slices(ci + 1)
                in_flight.append(pltpu.async_copy(
                    idx_hbm.at[bb1, pl.ds(rr1, _CROWS)],
                    idx_v.at[1 - buf], in_sems.at[1 - buf]))
            in_flight.pop(0).wait()
            if out_flight[buf] is not None:
                out_flight[buf].wait()

            ib = idx_v.at[buf]
            ob = out_v.at[buf]

            @plsc.parallel_loop(0, vecs, unroll=8)
            def vec_body(i):
                r = i // cols
                c = (i % cols) * _L
                iv = ib[r, pl.ds(c, _L)]
                ob[r, pl.ds(c, _L)] = plsc.load_gather(tbl_v, [iv])

            bb, rr = slices(ci)
            out_flight[buf] = pltpu.async_copy(
                out_v.at[buf], out_hbm.at[bb, pl.ds(rr, _CROWS)],
                out_sems.at[buf])

        for f in out_flight:
            if f is not None:
                f.wait()

    return _sc_gather


@jax.jit
def kernel(soft_label, pixel_weights):
    b, nc, h, wdim = soft_label.shape
    idx = _tc_argmax(soft_label)
    return _make_sc_gather(b, h, wdim)(pixel_weights, idx)


# speedup vs baseline: 1.2357x; 1.0479x over previous
import functools

import jax
import jax.numpy as jnp
from jax import lax
from jax.experimental import pallas as pl
from jax.experimental.pallas import tpu as pltpu
from jax.experimental.pallas import tpu_sc as plsc

_NUM_CLASSES = 19
_BH = 64
_PB = 4
_NC, _NS, _L = 2, 16, 16
_NW = _NC * _NS
_CROWS = 16


def _argmax_pack_body(x_ref, o_ref):
    x = x_ref[...]
    packed = None
    for bb in range(_PB):
        m = x[bb, 0]
        idx = jnp.zeros(m.shape, dtype=jnp.int32)
        for c in range(1, _NUM_CLASSES):
            v = x[bb, c]
            gt = v > m
            m = jnp.where(gt, v, m)
            idx = jnp.where(gt, c, idx)
        part = idx if bb == 0 else (idx << (8 * bb))
        packed = part if packed is None else (packed | part)
    o_ref[0] = packed


def _tc_argmax_packed(soft_label):
    b, nc, h, wdim = soft_label.shape
    return pl.pallas_call(
        _argmax_pack_body,
        grid=(b // _PB, h // _BH),
        in_specs=[pl.BlockSpec((_PB, nc, _BH, wdim),
                               lambda g, j: (g, 0, j, 0))],
        out_specs=pl.BlockSpec((1, _BH, wdim), lambda g, j: (g, j, 0)),
        out_shape=jax.ShapeDtypeStruct((b // _PB, h, wdim), jnp.int32),
    )(soft_label)


def _make_sc_gather(b, h, wdim):
    ng = b // _PB
    rows_per_w = (ng * h) // _NW
    n_chunks = rows_per_w // _CROWS
    vecs = (_CROWS * wdim) // _L
    cols = wdim // _L
    mesh = plsc.VectorSubcoreMesh(
        core_axis_name="c", subcore_axis_name="s",
        num_cores=_NC, num_subcores=_NS)

    @functools.partial(
        pl.kernel,
        mesh=mesh,
        compiler_params=pltpu.CompilerParams(
            needs_layout_passes=False, use_tc_tiling_on_sc=True),
        out_type=jax.ShapeDtypeStruct((b, h, wdim), jnp.float32),
        scratch_types=[
            pltpu.VMEM((_NUM_CLASSES,), jnp.float32),
            pltpu.VMEM((2, _CROWS, wdim), jnp.int32),
            pltpu.VMEM((2, _PB, _CROWS, wdim), jnp.float32),
            pltpu.SemaphoreType.DMA((2,)),
            pltpu.SemaphoreType.DMA((2,)),
        ],
    )
    def _sc_gather(tbl_hbm, pk_hbm, out_hbm, tbl_v, pk_v, out_v,
                   in_sems, out_sems):
        wid = lax.axis_index("s") * _NC + lax.axis_index("c")
        row_w = wid * rows_per_w

        def slices(ci):
            row = row_w + ci * _CROWS
            return row // h, row % h

        pltpu.sync_copy(tbl_hbm, tbl_v)

        in_flight = []
        g0, rr0 = slices(0)
        in_flight.append(pltpu.async_copy(
            pk_hbm.at[g0, pl.ds(rr0, _CROWS)], pk_v.at[0], in_sems.at[0]))

        out_flight = [[], []]
        for ci in range(n_chunks):
            buf = ci % 2
            if ci + 1 < n_chunks:
                g1, rr1 = slices(ci + 1)
                in_flight.append(pltpu.async_copy(
                    pk_hbm.at[g1, pl.ds(rr1, _CROWS)],
                    pk_v.at[1 - buf], in_sems.at[1 - buf]))
            in_flight.pop(0).wait()
            for f in out_flight[buf]:
                f.wait()
            out_flight[buf] = []

            ib = pk_v.at[buf]
            ob = out_v.at[buf]

            @plsc.parallel_loop(0, vecs, unroll=8)
            def vec_body(i):
                r = i // cols
                c = (i % cols) * _L
                w = ib[r, pl.ds(c, _L)]
                for bb in range(_PB):
                    iv = (w >> (8 * bb)) & 0xFF
                    ob[bb, r, pl.ds(c, _L)] = plsc.load_gather(tbl_v, [iv])

            g, rr = slices(ci)
            for bb in range(_PB):
                out_flight[buf].append(pltpu.async_copy(
                    out_v.at[buf, bb],
                    out_hbm.at[_PB * g + bb, pl.ds(rr, _CROWS)],
                    out_sems.at[buf]))

        for fl in out_flight:
            for f in fl:
                f.wait()

    return _sc_gather


@jax.jit
def kernel(soft_label, pixel_weights):
    b, nc, h, wdim = soft_label.shape
    pk = _tc_argmax_packed(soft_label)
    return _make_sc_gather(b, h, wdim)(pixel_weights, pk)
